# baseline (speedup 1.0000x reference)
import functools
import math

import numpy as np
import jax
import jax.numpy as jnp
from jax import lax
from jax.experimental import pallas as pl
from jax.experimental.pallas import tpu as pltpu
from jax.experimental.pallas import tpu_sc as plsc

_F32 = jnp.float32
_SQRT2 = math.sqrt(2.0)


def _gelu(x):
    return 0.5 * x * (1.0 + jax.lax.erf(x / _SQRT2))


def _tap(planes, nph, q2):
    off = 1 + (q2 // nph)
    return planes[q2 % nph][:, off:off + 512]


def _conv_gn_layer(planes, nph, w_ref, g_ref, b_ref):
    ys = []
    for p in range(nph // 2):
        x5 = jnp.concatenate([_tap(planes, nph, 2 * p + k - 2)
                              for k in range(5)], axis=0)
        ys.append(jnp.dot(w_ref[...], x5, preferred_element_type=_F32))
    y = jnp.concatenate(ys, axis=1) if len(ys) > 1 else ys[0]
    c, tt = y.shape
    yr = y.reshape(8, c // 8, tt)
    m = yr.mean(axis=(1, 2), keepdims=True)
    d = yr - m
    v = (d * d).mean(axis=(1, 2), keepdims=True)
    return _gelu((d * jax.lax.rsqrt(v + 1e-5)).reshape(c, tt)
                 * g_ref[...] + b_ref[...])


def _pad_planes(a, nph, c):
    z = jnp.zeros((c, 1), _F32)
    return [jnp.concatenate([z, a[:, p * 512:(p + 1) * 512], z], axis=1)
            for p in range(nph)]


def _stack_body(x_ref, w1_ref, g1_ref, b1_ref, w2_ref, g2_ref, b2_ref,
                w3_ref, g3_ref, b3_ref, out_ref, mean_ref, *, c0, c1, c2, c3):
    x = x_ref[0]
    xt = x.T
    x3 = xt.reshape(512, 8, c0)
    stack = jnp.concatenate([x3[:, q, :] for q in range(8)], axis=1)
    st = stack.T
    z = jnp.zeros((c0, 1), _F32)
    planes = [jnp.concatenate([z, st[q * c0:(q + 1) * c0, :], z], axis=1)
              for q in range(8)]
    a1 = _conv_gn_layer(planes, 8, w1_ref, g1_ref, b1_ref)
    a2 = _conv_gn_layer(_pad_planes(a1, 4, c1), 4,
                        w2_ref, g2_ref, b2_ref)
    a3 = _conv_gn_layer(_pad_planes(a2, 2, c2), 2,
                        w3_ref, g3_ref, b3_ref)
    out_ref[0] = a3.T
    mean_ref[0] = a3.mean(axis=-1).reshape(1, c3)


def _cnn_stack(x, layers):
    bsz, c0, _ = x.shape
    (w1, g1, b1), (w2, g2, b2), (w3, g3, b3) = layers
    c1, c2, c3 = w1.shape[0], w2.shape[0], w3.shape[0]
    wf = lambda w: jnp.concatenate([w[:, :, k] for k in range(5)], axis=1)
    rs = lambda v: v.reshape(-1, 1)
    cst = lambda shape: pl.BlockSpec(shape, lambda i: (0,) * len(shape))
    body = functools.partial(_stack_body, c0=c0, c1=c1, c2=c2, c3=c3)
    return pl.pallas_call(
        body,
        grid=(bsz,),
        in_specs=[
            pl.BlockSpec((1, c0, 4096), lambda i: (i, 0, 0)),
            cst((c1, 5 * c0)), cst((c1, 1)), cst((c1, 1)),
            cst((c2, 5 * c1)), cst((c2, 1)), cst((c2, 1)),
            cst((c3, 5 * c2)), cst((c3, 1)), cst((c3, 1)),
        ],
        out_specs=[pl.BlockSpec((1, 512, c3), lambda i: (i, 0, 0)),
                   pl.BlockSpec((1, 1, c3), lambda i: (i, 0, 0))],
        out_shape=[jax.ShapeDtypeStruct((bsz, 512, c3), _F32),
                   jax.ShapeDtypeStruct((bsz, 1, c3), _F32)],
    )(x, wf(w1), rs(g1), rs(b1), wf(w2), rs(g2), rs(b2), wf(w3), rs(g3),
      rs(b3))


def _gate_body(r_ref, w1_ref, b1_ref, lg_ref, lb_ref, w2_ref, b2_ref,
               gw_ref, gb_ref, wqt_ref, qcol_ref, bqcol_ref,
               logits_ref, qbd_ref):
    r = r_ref[...]
    x = jnp.dot(r, w1_ref[...], preferred_element_type=_F32) + b1_ref[...]
    m = x.mean(axis=-1, keepdims=True)
    d = x - m
    v = (d * d).mean(axis=-1, keepdims=True)
    x = _gelu(d * jax.lax.rsqrt(v + 1e-5) * lg_ref[...] + lb_ref[...])
    x = jnp.dot(x, w2_ref[...], preferred_element_type=_F32) + b2_ref[...]
    logits_ref[...] = (jnp.dot(x, gw_ref[...], preferred_element_type=_F32)
                       + gb_ref[...])
    qc = jnp.dot(wqt_ref[...], qcol_ref[...],
                 preferred_element_type=_F32) + bqcol_ref[...]
    dio = jax.lax.broadcasted_iota(jnp.int32, (192, 4), 0)
    hio = jax.lax.broadcasted_iota(jnp.int32, (192, 4), 1)
    qbd_ref[...] = jnp.where(dio // 48 == hio, qc, 0.0)


def _gate(r, p):
    bsz = r.shape[0]
    z2 = lambda i: (0, 0)
    full = lambda shape: pl.BlockSpec(shape, z2)
    return pl.pallas_call(
        _gate_body,
        grid=(1,),
        in_specs=[
            full((bsz, 192)),
            full((192, 64)), full((1, 64)), full((1, 64)), full((1, 64)),
            full((64, 32)), full((1, 32)),
            full((32, 8)), full((1, 8)),
            full((192, 192)), full((192, 1)), full((192, 1)),
        ],
        out_specs=[full((bsz, 8)), full((192, 4))],
        out_shape=[
            jax.ShapeDtypeStruct((bsz, 8), _F32),
            jax.ShapeDtypeStruct((192, 4), _F32),
        ],
    )(r, p['ctx_w1'], p['ctx_b1'].reshape(1, 64), p['ctx_lg'].reshape(1, 64),
      p['ctx_lb'].reshape(1, 64), p['ctx_w2'], p['ctx_b2'].reshape(1, 32),
      p['gate_w'], p['gate_b'].reshape(1, 8),
      p['ap_wq'].T, p['ap_q'].reshape(192, 1), p['ap_bq'].reshape(192, 1))


_NE = 8


def _sc_gate_body(lg_hbm, ti_hbm, tw_hbm, lg_v, ti_v, tw_v):
    wid = lax.axis_index("s") * 2 + lax.axis_index("c")

    @pl.when(wid == 0)
    def _():
        pltpu.sync_copy(lg_hbm, lg_v)
        for c in range(2):
            sl = pl.ds(c * 16, 16)
            vs = [lg_v[e, sl] for e in range(_NE)]
            mx = vs[0]
            for e in range(1, _NE):
                mx = jnp.maximum(mx, vs[e])
            exs = [jnp.exp(v - mx) for v in vs]
            tot = exs[0]
            for e in range(1, _NE):
                tot = tot + exs[e]
            ws = [ex / tot for ex in exs]
            m1 = ws[0]
            for e in range(1, _NE):
                m1 = jnp.maximum(m1, ws[e])
            i1 = jnp.full((16,), _NE, jnp.int32)
            for e in range(_NE - 1, -1, -1):
                i1 = jnp.where(ws[e] == m1, e, i1)
            ws2 = [jnp.where(i1 == e, -1.0, ws[e]) for e in range(_NE)]
            m2 = ws2[0]
            for e in range(1, _NE):
                m2 = jnp.maximum(m2, ws2[e])
            i2 = jnp.full((16,), _NE, jnp.int32)
            for e in range(_NE - 1, -1, -1):
                i2 = jnp.where(ws2[e] == m2, e, i2)
            denom = m1 + m2 + 1e-9
            ti_v[0, sl] = i1
            ti_v[1, sl] = i2
            tw_v[0, sl] = m1 / denom
            tw_v[1, sl] = m2 / denom
        pltpu.sync_copy(ti_v, ti_hbm)
        pltpu.sync_copy(tw_v, tw_hbm)


def _sc_gate(logits_t):
    return pl.kernel(
        _sc_gate_body,
        out_type=[jax.ShapeDtypeStruct((2, 32), jnp.int32),
                  jax.ShapeDtypeStruct((2, 32), _F32)],
        mesh=plsc.VectorSubcoreMesh(core_axis_name="c", subcore_axis_name="s"),
        scratch_types=[pltpu.VMEM((_NE, 32), _F32),
                       pltpu.VMEM((2, 32), jnp.int32),
                       pltpu.VMEM((2, 32), _F32)],
    )(logits_t)


_HEAD_E = np.repeat(np.eye(4, dtype=np.float32), 48, axis=1)
_INV_SQRT_DH = 1.0 / math.sqrt(48.0)


def _moe_body(topi_ref, h_ref, topw_ref, qbd_ref,
              w1a_ref, w1b_ref, w2a_ref, w2b_ref,
              b1a_ref, b1b_ref, b2a_ref, b2b_ref,
              wk_ref, bk_ref, wv_ref, bv_ref, eh_ref,
              wo_ref, bo_ref, pw_ref, pb_ref, out_ref):
    ht = h_ref[0]

    def expert(w1_ref, w2_ref, b1_ref, b2_ref):
        e1 = _gelu(jnp.dot(ht, w1_ref[0], preferred_element_type=_F32)
                   + b1_ref[0])
        return jnp.dot(e1, w2_ref[0], preferred_element_type=_F32) + b2_ref[0]

    e2a = expert(w1a_ref, w2a_ref, b1a_ref, b2a_ref)
    e2b = expert(w1b_ref, w2b_ref, b1b_ref, b2b_ref)
    hm = ht + topw_ref[0, 0, 0] * e2a + topw_ref[0, 0, 1] * e2b

    kx = jnp.dot(hm, wk_ref[...], preferred_element_type=_F32) + bk_ref[...]
    sc = jnp.dot(kx, qbd_ref[...],
                 preferred_element_type=_F32) * _INV_SQRT_DH
    mx = sc.max(axis=0, keepdims=True)
    a = jnp.exp(sc - mx)
    a = a / a.sum(axis=0, keepdims=True)
    af = jnp.dot(a, eh_ref[...], preferred_element_type=_F32)
    vx = jnp.dot(hm, wv_ref[...], preferred_element_type=_F32) + bv_ref[...]
    pooled = jnp.sum(af * vx, axis=0, keepdims=True)
    ov = jnp.dot(pooled, wo_ref[...], preferred_element_type=_F32) + bo_ref[...]
    z = jnp.dot(ov, pw_ref[...], preferred_element_type=_F32) + pb_ref[...]
    z = z / (jnp.sqrt(jnp.sum(z * z)) + 1e-12)
    out_ref[0] = z


def _moe_attn(h_t, topi, topw, qbd, p):
    bsz = h_t.shape[0]
    w1 = p['exp_w1']
    w2 = p['exp_w2']
    b1 = p['exp_b1'].reshape(8, 1, 192)
    b2 = p['exp_b2'].reshape(8, 1, 192)
    topw3 = topw.reshape(bsz, 1, 2)

    def fixed(shape):
        nd = len(shape)
        return pl.BlockSpec(shape, lambda i, s, _n=nd: (0,) * _n)

    grid_spec = pltpu.PrefetchScalarGridSpec(
        num_scalar_prefetch=1,
        grid=(bsz,),
        in_specs=[
            pl.BlockSpec((1, 512, 192), lambda i, s: (i, 0, 0)),
            pl.BlockSpec((1, 1, 2), lambda i, s: (i, 0, 0)),
            fixed((192, 4)),
            pl.BlockSpec((1, 192, 192), lambda i, s: (s[i, 0], 0, 0)),
            pl.BlockSpec((1, 192, 192), lambda i, s: (s[i, 1], 0, 0)),
            pl.BlockSpec((1, 192, 192), lambda i, s: (s[i, 0], 0, 0)),
            pl.BlockSpec((1, 192, 192), lambda i, s: (s[i, 1], 0, 0)),
            pl.BlockSpec((1, 1, 192), lambda i, s: (s[i, 0], 0, 0)),
            pl.BlockSpec((1, 1, 192), lambda i, s: (s[i, 1], 0, 0)),
            pl.BlockSpec((1, 1, 192), lambda i, s: (s[i, 0], 0, 0)),
            pl.BlockSpec((1, 1, 192), lambda i, s: (s[i, 1], 0, 0)),
            fixed((192, 192)), fixed((1, 192)),
            fixed((192, 192)), fixed((1, 192)),
            fixed((4, 192)),
            fixed((192, 192)), fixed((1, 192)),
            fixed((192, 128)), fixed((1, 128)),
        ],
        out_specs=pl.BlockSpec((1, 1, 128), lambda i, s: (i, 0, 0)),
    )
    out = pl.pallas_call(
        _moe_body,
        grid_spec=grid_spec,
        out_shape=jax.ShapeDtypeStruct((bsz, 1, 128), _F32),
    )(topi, h_t, topw3, qbd,
      w1, w1, w2, w2, b1, b1, b2, b2,
      p['ap_wk'], p['ap_bk'].reshape(1, 192),
      p['ap_wv'], p['ap_bv'].reshape(1, 192),
      jnp.asarray(_HEAD_E),
      p['ap_wo'], p['ap_bo'].reshape(1, 192),
      p['proj_w'], p['proj_b'].reshape(1, 128))
    return out.reshape(bsz, 128)


def kernel(x_emg, x_imu, params):
    p = params
    he_t, me = _cnn_stack(x_emg, p['emg'])
    hi_t, mi = _cnn_stack(x_imu, p['imu'])
    r = jnp.concatenate([me[:, 0, :], mi[:, 0, :]], axis=-1)
    h_t = jnp.concatenate([he_t, hi_t], axis=-1)
    logits, qbd = _gate(r, p)
    ti_t, tw_t = _sc_gate(logits.T)
    return _moe_attn(h_t, ti_t.T, tw_t.T, qbd, p)


# speedup vs baseline: 1.1082x; 1.1082x over previous
import functools
import math

import numpy as np
import jax
import jax.numpy as jnp
from jax import lax
from jax.experimental import pallas as pl
from jax.experimental.pallas import tpu as pltpu
from jax.experimental.pallas import tpu_sc as plsc

_F32 = jnp.float32
_SQRT2 = math.sqrt(2.0)


def _gelu(x):
    return 0.5 * x * (1.0 + jax.lax.erf(x / _SQRT2))


def _tap(planes, nph, q2):
    off = 1 + (q2 // nph)
    return planes[q2 % nph][:, off:off + 512]


def _zpad(a):
    z = jnp.zeros((a.shape[0], 1), _F32)
    return jnp.concatenate([z, a, z], axis=1)


def _conv_gn_layer(planes, nph, w_ref, g_ref, b_ref, pad_out):
    nout = nph // 2
    ys = [jnp.dot(w_ref[...],
                  jnp.concatenate([_tap(planes, nph, 2 * p + k - 2)
                                   for k in range(5)], axis=0),
                  preferred_element_type=_F32)
          for p in range(nout)]
    c = ys[0].shape[0]
    cg = c // 8
    n = float(nout * 512 * cg)
    s1 = ys[0].reshape(8, cg, 512).sum(axis=(1, 2), keepdims=True)
    for yp in ys[1:]:
        s1 = s1 + yp.reshape(8, cg, 512).sum(axis=(1, 2), keepdims=True)
    m = s1 / n
    ds = [yp.reshape(8, cg, 512) - m for yp in ys]
    s2 = ds[0]
    s2 = (s2 * s2).sum(axis=(1, 2), keepdims=True)
    for d in ds[1:]:
        s2 = s2 + (d * d).sum(axis=(1, 2), keepdims=True)
    inv = jax.lax.rsqrt(s2 / n + 1e-5)
    outs = []
    for d in ds:
        a = _gelu((d * inv).reshape(c, 512) * g_ref[...] + b_ref[...])
        outs.append(_zpad(a) if pad_out else a)
    return outs


def _stack_body(x_ref, w1_ref, g1_ref, b1_ref, w2_ref, g2_ref, b2_ref,
                w3_ref, g3_ref, b3_ref, out_ref, mean_ref, *, c0, c1, c2, c3):
    x = x_ref[0]
    xt = x.T
    x3 = xt.reshape(512, 8, c0)
    planes = [_zpad(x3[:, q, :].T) for q in range(8)]
    p1 = _conv_gn_layer(planes, 8, w1_ref, g1_ref, b1_ref, True)
    p2 = _conv_gn_layer(p1, 4, w2_ref, g2_ref, b2_ref, True)
    a3 = _conv_gn_layer(p2, 2, w3_ref, g3_ref, b3_ref, False)[0]
    out_ref[0] = a3.T
    mean_ref[0] = a3.mean(axis=-1).reshape(1, c3)


def _cnn_stack(x, layers):
    bsz, c0, _ = x.shape
    (w1, g1, b1), (w2, g2, b2), (w3, g3, b3) = layers
    c1, c2, c3 = w1.shape[0], w2.shape[0], w3.shape[0]
    wf = lambda w: jnp.concatenate([w[:, :, k] for k in range(5)], axis=1)
    rs = lambda v: v.reshape(-1, 1)
    cst = lambda shape: pl.BlockSpec(shape, lambda i: (0,) * len(shape))
    body = functools.partial(_stack_body, c0=c0, c1=c1, c2=c2, c3=c3)
    return pl.pallas_call(
        body,
        grid=(bsz,),
        in_specs=[
            pl.BlockSpec((1, c0, 4096), lambda i: (i, 0, 0)),
            cst((c1, 5 * c0)), cst((c1, 1)), cst((c1, 1)),
            cst((c2, 5 * c1)), cst((c2, 1)), cst((c2, 1)),
            cst((c3, 5 * c2)), cst((c3, 1)), cst((c3, 1)),
        ],
        out_specs=[pl.BlockSpec((1, 512, c3), lambda i: (i, 0, 0)),
                   pl.BlockSpec((1, 1, c3), lambda i: (i, 0, 0))],
        out_shape=[jax.ShapeDtypeStruct((bsz, 512, c3), _F32),
                   jax.ShapeDtypeStruct((bsz, 1, c3), _F32)],
    )(x, wf(w1), rs(g1), rs(b1), wf(w2), rs(g2), rs(b2), wf(w3), rs(g3),
      rs(b3))


def _gate_body(r_ref, w1_ref, b1_ref, lg_ref, lb_ref, w2_ref, b2_ref,
               gw_ref, gb_ref, wqt_ref, qcol_ref, bqcol_ref,
               wk_ref, bk_ref, logits_ref, wkq_ref, bkq_ref):
    r = r_ref[...]
    x = jnp.dot(r, w1_ref[...], preferred_element_type=_F32) + b1_ref[...]
    m = x.mean(axis=-1, keepdims=True)
    d = x - m
    v = (d * d).mean(axis=-1, keepdims=True)
    x = _gelu(d * jax.lax.rsqrt(v + 1e-5) * lg_ref[...] + lb_ref[...])
    x = jnp.dot(x, w2_ref[...], preferred_element_type=_F32) + b2_ref[...]
    logits_ref[...] = (jnp.dot(x, gw_ref[...], preferred_element_type=_F32)
                       + gb_ref[...])
    qc = jnp.dot(wqt_ref[...], qcol_ref[...],
                 preferred_element_type=_F32) + bqcol_ref[...]
    dio = jax.lax.broadcasted_iota(jnp.int32, (192, 4), 0)
    hio = jax.lax.broadcasted_iota(jnp.int32, (192, 4), 1)
    qbd = jnp.where(dio // 48 == hio, qc, 0.0)
    wkq_ref[...] = jnp.dot(wk_ref[...], qbd, preferred_element_type=_F32)
    bkq_ref[...] = jnp.dot(bk_ref[...], qbd, preferred_element_type=_F32)


def _gate(r, p):
    bsz = r.shape[0]
    z2 = lambda i: (0, 0)
    full = lambda shape: pl.BlockSpec(shape, z2)
    return pl.pallas_call(
        _gate_body,
        grid=(1,),
        in_specs=[
            full((bsz, 192)),
            full((192, 64)), full((1, 64)), full((1, 64)), full((1, 64)),
            full((64, 32)), full((1, 32)),
            full((32, 8)), full((1, 8)),
            full((192, 192)), full((192, 1)), full((192, 1)),
            full((192, 192)), full((1, 192)),
        ],
        out_specs=[full((bsz, 8)), full((192, 4)), full((1, 4))],
        out_shape=[
            jax.ShapeDtypeStruct((bsz, 8), _F32),
            jax.ShapeDtypeStruct((192, 4), _F32),
            jax.ShapeDtypeStruct((1, 4), _F32),
        ],
    )(r, p['ctx_w1'], p['ctx_b1'].reshape(1, 64), p['ctx_lg'].reshape(1, 64),
      p['ctx_lb'].reshape(1, 64), p['ctx_w2'], p['ctx_b2'].reshape(1, 32),
      p['gate_w'], p['gate_b'].reshape(1, 8),
      p['ap_wq'].T, p['ap_q'].reshape(192, 1), p['ap_bq'].reshape(192, 1),
      p['ap_wk'], p['ap_bk'].reshape(1, 192))


_NE = 8


def _sc_gate_body(lg_hbm, ti_hbm, tw_hbm, lg_v, ti_v, tw_v):
    wid = lax.axis_index("s") * 2 + lax.axis_index("c")

    @pl.when(wid == 0)
    def _():
        pltpu.sync_copy(lg_hbm, lg_v)
        for c in range(2):
            sl = pl.ds(c * 16, 16)
            vs = [lg_v[e, sl] for e in range(_NE)]
            mx = vs[0]
            for e in range(1, _NE):
                mx = jnp.maximum(mx, vs[e])
            exs = [jnp.exp(v - mx) for v in vs]
            tot = exs[0]
            for e in range(1, _NE):
                tot = tot + exs[e]
            ws = [ex / tot for ex in exs]
            m1 = ws[0]
            for e in range(1, _NE):
                m1 = jnp.maximum(m1, ws[e])
            i1 = jnp.full((16,), _NE, jnp.int32)
            for e in range(_NE - 1, -1, -1):
                i1 = jnp.where(ws[e] == m1, e, i1)
            ws2 = [jnp.where(i1 == e, -1.0, ws[e]) for e in range(_NE)]
            m2 = ws2[0]
            for e in range(1, _NE):
                m2 = jnp.maximum(m2, ws2[e])
            i2 = jnp.full((16,), _NE, jnp.int32)
            for e in range(_NE - 1, -1, -1):
                i2 = jnp.where(ws2[e] == m2, e, i2)
            denom = m1 + m2 + 1e-9
            ti_v[0, sl] = i1
            ti_v[1, sl] = i2
            tw_v[0, sl] = m1 / denom
            tw_v[1, sl] = m2 / denom
        pltpu.sync_copy(ti_v, ti_hbm)
        pltpu.sync_copy(tw_v, tw_hbm)


def _sc_gate(logits_t):
    return pl.kernel(
        _sc_gate_body,
        out_type=[jax.ShapeDtypeStruct((2, 32), jnp.int32),
                  jax.ShapeDtypeStruct((2, 32), _F32)],
        mesh=plsc.VectorSubcoreMesh(core_axis_name="c", subcore_axis_name="s"),
        scratch_types=[pltpu.VMEM((_NE, 32), _F32),
                       pltpu.VMEM((2, 32), jnp.int32),
                       pltpu.VMEM((2, 32), _F32)],
    )(logits_t)


_HEAD_E = np.repeat(np.eye(4, dtype=np.float32), 48, axis=1)
_INV_SQRT_DH = 1.0 / math.sqrt(48.0)


def _moe_body(topi_ref, he_ref, hi_ref, topw_ref, wkq_ref, bkq_ref,
              w1a_ref, w1b_ref, w2a_ref, w2b_ref,
              b1a_ref, b1b_ref, b2a_ref, b2b_ref,
              wv_ref, bv_ref, eh_ref,
              wo_ref, bo_ref, pw_ref, pb_ref, out_ref):
    ht = jnp.concatenate([he_ref[0], hi_ref[0]], axis=1)

    def expert(w1_ref, w2_ref, b1_ref, b2_ref):
        e1 = _gelu(jnp.dot(ht, w1_ref[0], preferred_element_type=_F32)
                   + b1_ref[0])
        return jnp.dot(e1, w2_ref[0], preferred_element_type=_F32) + b2_ref[0]

    e2a = expert(w1a_ref, w2a_ref, b1a_ref, b2a_ref)
    e2b = expert(w1b_ref, w2b_ref, b1b_ref, b2b_ref)
    hm = ht + topw_ref[0, 0, 0] * e2a + topw_ref[0, 0, 1] * e2b

    sc = (jnp.dot(hm, wkq_ref[...], preferred_element_type=_F32)
          + bkq_ref[...]) * _INV_SQRT_DH
    mx = sc.max(axis=0, keepdims=True)
    a = jnp.exp(sc - mx)
    a = a / a.sum(axis=0, keepdims=True)
    af = jnp.dot(a, eh_ref[...], preferred_element_type=_F32)
    vx = jnp.dot(hm, wv_ref[...], preferred_element_type=_F32) + bv_ref[...]
    pooled = jnp.sum(af * vx, axis=0, keepdims=True)
    ov = jnp.dot(pooled, wo_ref[...], preferred_element_type=_F32) + bo_ref[...]
    z = jnp.dot(ov, pw_ref[...], preferred_element_type=_F32) + pb_ref[...]
    z = z / (jnp.sqrt(jnp.sum(z * z)) + 1e-12)
    out_ref[0] = z


def _moe_attn(he_t, hi_t, topi, topw, wkq, bkq, p):
    bsz = he_t.shape[0]
    w1 = p['exp_w1']
    w2 = p['exp_w2']
    b1 = p['exp_b1'].reshape(8, 1, 192)
    b2 = p['exp_b2'].reshape(8, 1, 192)
    topw3 = topw.reshape(bsz, 1, 2)

    def fixed(shape):
        nd = len(shape)
        return pl.BlockSpec(shape, lambda i, s, _n=nd: (0,) * _n)

    grid_spec = pltpu.PrefetchScalarGridSpec(
        num_scalar_prefetch=1,
        grid=(bsz,),
        in_specs=[
            pl.BlockSpec((1, 512, 128), lambda i, s: (i, 0, 0)),
            pl.BlockSpec((1, 512, 64), lambda i, s: (i, 0, 0)),
            pl.BlockSpec((1, 1, 2), lambda i, s: (i, 0, 0)),
            fixed((192, 4)), fixed((1, 4)),
            pl.BlockSpec((1, 192, 192), lambda i, s: (s[i, 0], 0, 0)),
            pl.BlockSpec((1, 192, 192), lambda i, s: (s[i, 1], 0, 0)),
            pl.BlockSpec((1, 192, 192), lambda i, s: (s[i, 0], 0, 0)),
            pl.BlockSpec((1, 192, 192), lambda i, s: (s[i, 1], 0, 0)),
            pl.BlockSpec((1, 1, 192), lambda i, s: (s[i, 0], 0, 0)),
            pl.BlockSpec((1, 1, 192), lambda i, s: (s[i, 1], 0, 0)),
            pl.BlockSpec((1, 1, 192), lambda i, s: (s[i, 0], 0, 0)),
            pl.BlockSpec((1, 1, 192), lambda i, s: (s[i, 1], 0, 0)),
            fixed((192, 192)), fixed((1, 192)),
            fixed((4, 192)),
            fixed((192, 192)), fixed((1, 192)),
            fixed((192, 128)), fixed((1, 128)),
        ],
        out_specs=pl.BlockSpec((1, 1, 128), lambda i, s: (i, 0, 0)),
    )
    out = pl.pallas_call(
        _moe_body,
        grid_spec=grid_spec,
        out_shape=jax.ShapeDtypeStruct((bsz, 1, 128), _F32),
    )(topi, he_t, hi_t, topw3, wkq, bkq,
      w1, w1, w2, w2, b1, b1, b2, b2,
      p['ap_wv'], p['ap_bv'].reshape(1, 192),
      jnp.asarray(_HEAD_E),
      p['ap_wo'], p['ap_bo'].reshape(1, 192),
      p['proj_w'], p['proj_b'].reshape(1, 128))
    return out.reshape(bsz, 128)


def kernel(x_emg, x_imu, params):
    p = params
    he_t, me = _cnn_stack(x_emg, p['emg'])
    hi_t, mi = _cnn_stack(x_imu, p['imu'])
    r = jnp.concatenate([me[:, 0, :], mi[:, 0, :]], axis=-1)
    logits, wkq, bkq = _gate(r, p)
    ti_t, tw_t = _sc_gate(logits.T)
    return _moe_attn(he_t, hi_t, ti_t.T, tw_t.T, wkq, bkq, p)


# speedup vs baseline: 1.1155x; 1.0066x over previous
import functools
import math

import numpy as np
import jax
import jax.numpy as jnp
from jax import lax
from jax.experimental import pallas as pl
from jax.experimental.pallas import tpu as pltpu
from jax.experimental.pallas import tpu_sc as plsc

_F32 = jnp.float32
_SQRT2 = math.sqrt(2.0)


def _gelu(x):
    return 0.5 * x * (1.0 + jax.lax.erf(x / _SQRT2))


def _tap(planes, nph, q2):
    off = 1 + (q2 // nph)
    return planes[q2 % nph][:, off:off + 512]


def _zpad(a):
    z = jnp.zeros((a.shape[0], 1), _F32)
    return jnp.concatenate([z, a, z], axis=1)


def _conv_gn_layer(planes, nph, w_ref, g_ref, b_ref, pad_out):
    nout = nph // 2
    ys = [jnp.dot(w_ref[...],
                  jnp.concatenate([_tap(planes, nph, 2 * p + k - 2)
                                   for k in range(5)], axis=0),
                  preferred_element_type=_F32)
          for p in range(nout)]
    c = ys[0].shape[0]
    cg = c // 8
    n = float(nout * 512 * cg)
    s1 = ys[0].reshape(8, cg, 512).sum(axis=(1, 2), keepdims=True)
    for yp in ys[1:]:
        s1 = s1 + yp.reshape(8, cg, 512).sum(axis=(1, 2), keepdims=True)
    m = s1 / n
    ds = [yp.reshape(8, cg, 512) - m for yp in ys]
    s2 = ds[0]
    s2 = (s2 * s2).sum(axis=(1, 2), keepdims=True)
    for d in ds[1:]:
        s2 = s2 + (d * d).sum(axis=(1, 2), keepdims=True)
    inv = jax.lax.rsqrt(s2 / n + 1e-5)
    outs = []
    for d in ds:
        a = _gelu((d * inv).reshape(c, 512) * g_ref[...] + b_ref[...])
        outs.append(_zpad(a) if pad_out else a)
    return outs


def _run_stack(x, w1_ref, g1_ref, b1_ref, w2_ref, g2_ref, b2_ref,
               w3_ref, g3_ref, b3_ref):
    c0 = x.shape[0]
    xt = x.T
    x3 = xt.reshape(512, 8, c0)
    planes = [_zpad(x3[:, q, :].T) for q in range(8)]
    p1 = _conv_gn_layer(planes, 8, w1_ref, g1_ref, b1_ref, True)
    p2 = _conv_gn_layer(p1, 4, w2_ref, g2_ref, b2_ref, True)
    return _conv_gn_layer(p2, 2, w3_ref, g3_ref, b3_ref, False)[0]


def _stacks_body(xe_ref, xi_ref,
                 ew1_ref, eg1_ref, eb1_ref, ew2_ref, eg2_ref, eb2_ref,
                 ew3_ref, eg3_ref, eb3_ref,
                 iw1_ref, ig1_ref, ib1_ref, iw2_ref, ig2_ref, ib2_ref,
                 iw3_ref, ig3_ref, ib3_ref,
                 oute_ref, meane_ref, outi_ref, meani_ref):
    ae = _run_stack(xe_ref[0], ew1_ref, eg1_ref, eb1_ref, ew2_ref, eg2_ref,
                    eb2_ref, ew3_ref, eg3_ref, eb3_ref)
    ai = _run_stack(xi_ref[0], iw1_ref, ig1_ref, ib1_ref, iw2_ref, ig2_ref,
                    ib2_ref, iw3_ref, ig3_ref, ib3_ref)
    oute_ref[0] = ae.T
    meane_ref[0] = ae.mean(axis=-1).reshape(1, ae.shape[0])
    outi_ref[0] = ai.T
    meani_ref[0] = ai.mean(axis=-1).reshape(1, ai.shape[0])


def _cnn_stacks(x_emg, x_imu, elayers, ilayers):
    bsz = x_emg.shape[0]
    wf = lambda w: jnp.concatenate([w[:, :, k] for k in range(5)], axis=1)
    rs = lambda v: v.reshape(-1, 1)
    cst = lambda shape: pl.BlockSpec(shape, lambda i: (0,) * len(shape))

    def wspecs(layers):
        specs, args = [], []
        cin = layers[0][0].shape[1]
        for (w, g, b) in layers:
            cout = w.shape[0]
            specs += [cst((cout, 5 * cin)), cst((cout, 1)), cst((cout, 1))]
            args += [wf(w), rs(g), rs(b)]
            cin = cout
        return specs, args

    especs, eargs = wspecs(elayers)
    ispecs, iargs = wspecs(ilayers)
    return pl.pallas_call(
        _stacks_body,
        grid=(bsz,),
        in_specs=[pl.BlockSpec((1, 16, 4096), lambda i: (i, 0, 0)),
                  pl.BlockSpec((1, 72, 4096), lambda i: (i, 0, 0))]
                 + especs + ispecs,
        out_specs=[pl.BlockSpec((1, 512, 128), lambda i: (i, 0, 0)),
                   pl.BlockSpec((1, 1, 128), lambda i: (i, 0, 0)),
                   pl.BlockSpec((1, 512, 64), lambda i: (i, 0, 0)),
                   pl.BlockSpec((1, 1, 64), lambda i: (i, 0, 0))],
        out_shape=[jax.ShapeDtypeStruct((bsz, 512, 128), _F32),
                   jax.ShapeDtypeStruct((bsz, 1, 128), _F32),
                   jax.ShapeDtypeStruct((bsz, 512, 64), _F32),
                   jax.ShapeDtypeStruct((bsz, 1, 64), _F32)],
    )(x_emg, x_imu, *eargs, *iargs)


def _gate_body(r_ref, w1_ref, b1_ref, lg_ref, lb_ref, w2_ref, b2_ref,
               gw_ref, gb_ref, wqt_ref, qcol_ref, bqcol_ref,
               wk_ref, bk_ref, logits_ref, wkq_ref, bkq_ref):
    r = r_ref[...]
    x = jnp.dot(r, w1_ref[...], preferred_element_type=_F32) + b1_ref[...]
    m = x.mean(axis=-1, keepdims=True)
    d = x - m
    v = (d * d).mean(axis=-1, keepdims=True)
    x = _gelu(d * jax.lax.rsqrt(v + 1e-5) * lg_ref[...] + lb_ref[...])
    x = jnp.dot(x, w2_ref[...], preferred_element_type=_F32) + b2_ref[...]
    logits_ref[...] = (jnp.dot(x, gw_ref[...], preferred_element_type=_F32)
                       + gb_ref[...])
    qc = jnp.dot(wqt_ref[...], qcol_ref[...],
                 preferred_element_type=_F32) + bqcol_ref[...]
    dio = jax.lax.broadcasted_iota(jnp.int32, (192, 4), 0)
    hio = jax.lax.broadcasted_iota(jnp.int32, (192, 4), 1)
    qbd = jnp.where(dio // 48 == hio, qc, 0.0)
    wkq_ref[...] = jnp.dot(wk_ref[...], qbd, preferred_element_type=_F32)
    bkq_ref[...] = jnp.dot(bk_ref[...], qbd, preferred_element_type=_F32)


def _gate(r, p):
    bsz = r.shape[0]
    z2 = lambda i: (0, 0)
    full = lambda shape: pl.BlockSpec(shape, z2)
    return pl.pallas_call(
        _gate_body,
        grid=(1,),
        in_specs=[
            full((bsz, 192)),
            full((192, 64)), full((1, 64)), full((1, 64)), full((1, 64)),
            full((64, 32)), full((1, 32)),
            full((32, 8)), full((1, 8)),
            full((192, 192)), full((192, 1)), full((192, 1)),
            full((192, 192)), full((1, 192)),
        ],
        out_specs=[full((bsz, 8)), full((192, 4)), full((1, 4))],
        out_shape=[
            jax.ShapeDtypeStruct((bsz, 8), _F32),
            jax.ShapeDtypeStruct((192, 4), _F32),
            jax.ShapeDtypeStruct((1, 4), _F32),
        ],
    )(r, p['ctx_w1'], p['ctx_b1'].reshape(1, 64), p['ctx_lg'].reshape(1, 64),
      p['ctx_lb'].reshape(1, 64), p['ctx_w2'], p['ctx_b2'].reshape(1, 32),
      p['gate_w'], p['gate_b'].reshape(1, 8),
      p['ap_wq'].T, p['ap_q'].reshape(192, 1), p['ap_bq'].reshape(192, 1),
      p['ap_wk'], p['ap_bk'].reshape(1, 192))


_NE = 8


def _sc_gate_body(lg_hbm, ti_hbm, tw_hbm, lg_v, ti_v, tw_v):
    wid = lax.axis_index("s") * 2 + lax.axis_index("c")

    @pl.when(wid == 0)
    def _():
        pltpu.sync_copy(lg_hbm, lg_v)
        for c in range(2):
            sl = pl.ds(c * 16, 16)
            vs = [lg_v[e, sl] for e in range(_NE)]
            mx = vs[0]
            for e in range(1, _NE):
                mx = jnp.maximum(mx, vs[e])
            exs = [jnp.exp(v - mx) for v in vs]
            tot = exs[0]
            for e in range(1, _NE):
                tot = tot + exs[e]
            ws = [ex / tot for ex in exs]
            m1 = ws[0]
            for e in range(1, _NE):
                m1 = jnp.maximum(m1, ws[e])
            i1 = jnp.full((16,), _NE, jnp.int32)
            for e in range(_NE - 1, -1, -1):
                i1 = jnp.where(ws[e] == m1, e, i1)
            ws2 = [jnp.where(i1 == e, -1.0, ws[e]) for e in range(_NE)]
            m2 = ws2[0]
            for e in range(1, _NE):
                m2 = jnp.maximum(m2, ws2[e])
            i2 = jnp.full((16,), _NE, jnp.int32)
            for e in range(_NE - 1, -1, -1):
                i2 = jnp.where(ws2[e] == m2, e, i2)
            denom = m1 + m2 + 1e-9
            ti_v[0, sl] = i1
            ti_v[1, sl] = i2
            tw_v[0, sl] = m1 / denom
            tw_v[1, sl] = m2 / denom
        pltpu.sync_copy(ti_v, ti_hbm)
        pltpu.sync_copy(tw_v, tw_hbm)


def _sc_gate(logits_t):
    return pl.kernel(
        _sc_gate_body,
        out_type=[jax.ShapeDtypeStruct((2, 32), jnp.int32),
                  jax.ShapeDtypeStruct((2, 32), _F32)],
        mesh=plsc.VectorSubcoreMesh(core_axis_name="c", subcore_axis_name="s"),
        scratch_types=[pltpu.VMEM((_NE, 32), _F32),
                       pltpu.VMEM((2, 32), jnp.int32),
                       pltpu.VMEM((2, 32), _F32)],
    )(logits_t)


_HEAD_E = np.repeat(np.eye(4, dtype=np.float32), 48, axis=1)
_INV_SQRT_DH = 1.0 / math.sqrt(48.0)


def _moe_body(topi_ref, he_ref, hi_ref, topw_ref, wkq_ref, bkq_ref,
              w1a_ref, w1b_ref, w2a_ref, w2b_ref,
              b1a_ref, b1b_ref, b2a_ref, b2b_ref,
              wv_ref, bv_ref, eh_ref,
              wo_ref, bo_ref, pw_ref, pb_ref, out_ref):
    ht = jnp.concatenate([he_ref[0], hi_ref[0]], axis=1)
    htb = ht.astype(jnp.bfloat16)

    def expert(w1_ref, w2_ref, b1_ref, b2_ref):
        e1 = _gelu(jnp.dot(htb, w1_ref[0].astype(jnp.bfloat16),
                           preferred_element_type=_F32) + b1_ref[0])
        return jnp.dot(e1.astype(jnp.bfloat16),
                       w2_ref[0].astype(jnp.bfloat16),
                       preferred_element_type=_F32) + b2_ref[0]

    e2a = expert(w1a_ref, w2a_ref, b1a_ref, b2a_ref)
    e2b = expert(w1b_ref, w2b_ref, b1b_ref, b2b_ref)
    hm = ht + topw_ref[0, 0, 0] * e2a + topw_ref[0, 0, 1] * e2b

    sc = (jnp.dot(hm, wkq_ref[...], preferred_element_type=_F32)
          + bkq_ref[...]) * _INV_SQRT_DH
    mx = sc.max(axis=0, keepdims=True)
    a = jnp.exp(sc - mx)
    a = a / a.sum(axis=0, keepdims=True)
    af = jnp.dot(a, eh_ref[...], preferred_element_type=_F32)
    vx = jnp.dot(hm, wv_ref[...], preferred_element_type=_F32) + bv_ref[...]
    pooled = jnp.sum(af * vx, axis=0, keepdims=True)
    ov = jnp.dot(pooled, wo_ref[...], preferred_element_type=_F32) + bo_ref[...]
    z = jnp.dot(ov, pw_ref[...], preferred_element_type=_F32) + pb_ref[...]
    z = z / (jnp.sqrt(jnp.sum(z * z)) + 1e-12)
    out_ref[0] = z


def _moe_attn(he_t, hi_t, topi, topw, wkq, bkq, p):
    bsz = he_t.shape[0]
    w1 = p['exp_w1']
    w2 = p['exp_w2']
    b1 = p['exp_b1'].reshape(8, 1, 192)
    b2 = p['exp_b2'].reshape(8, 1, 192)
    topw3 = topw.reshape(bsz, 1, 2)

    def fixed(shape):
        nd = len(shape)
        return pl.BlockSpec(shape, lambda i, s, _n=nd: (0,) * _n)

    grid_spec = pltpu.PrefetchScalarGridSpec(
        num_scalar_prefetch=1,
        grid=(bsz,),
        in_specs=[
            pl.BlockSpec((1, 512, 128), lambda i, s: (i, 0, 0)),
            pl.BlockSpec((1, 512, 64), lambda i, s: (i, 0, 0)),
            pl.BlockSpec((1, 1, 2), lambda i, s: (i, 0, 0)),
            fixed((192, 4)), fixed((1, 4)),
            pl.BlockSpec((1, 192, 192), lambda i, s: (s[i, 0], 0, 0)),
            pl.BlockSpec((1, 192, 192), lambda i, s: (s[i, 1], 0, 0)),
            pl.BlockSpec((1, 192, 192), lambda i, s: (s[i, 0], 0, 0)),
            pl.BlockSpec((1, 192, 192), lambda i, s: (s[i, 1], 0, 0)),
            pl.BlockSpec((1, 1, 192), lambda i, s: (s[i, 0], 0, 0)),
            pl.BlockSpec((1, 1, 192), lambda i, s: (s[i, 1], 0, 0)),
            pl.BlockSpec((1, 1, 192), lambda i, s: (s[i, 0], 0, 0)),
            pl.BlockSpec((1, 1, 192), lambda i, s: (s[i, 1], 0, 0)),
            fixed((192, 192)), fixed((1, 192)),
            fixed((4, 192)),
            fixed((192, 192)), fixed((1, 192)),
            fixed((192, 128)), fixed((1, 128)),
        ],
        out_specs=pl.BlockSpec((1, 1, 128), lambda i, s: (i, 0, 0)),
    )
    out = pl.pallas_call(
        _moe_body,
        grid_spec=grid_spec,
        out_shape=jax.ShapeDtypeStruct((bsz, 1, 128), _F32),
    )(topi, he_t, hi_t, topw3, wkq, bkq,
      w1, w1, w2, w2, b1, b1, b2, b2,
      p['ap_wv'], p['ap_bv'].reshape(1, 192),
      jnp.asarray(_HEAD_E),
      p['ap_wo'], p['ap_bo'].reshape(1, 192),
      p['proj_w'], p['proj_b'].reshape(1, 128))
    return out.reshape(bsz, 128)


def kernel(x_emg, x_imu, params):
    p = params
    he_t, me, hi_t, mi = _cnn_stacks(x_emg, x_imu, p['emg'], p['imu'])
    r = jnp.concatenate([me[:, 0, :], mi[:, 0, :]], axis=-1)
    logits, wkq, bkq = _gate(r, p)
    ti_t, tw_t = _sc_gate(logits.T)
    return _moe_attn(he_t, hi_t, ti_t.T, tw_t.T, wkq, bkq, p)
